# Initial kernel scaffold; baseline (speedup 1.0000x reference)
import jax
import jax.numpy as jnp
from jax.experimental import pallas as pl


def kernel(x, edge_index, W1, b1, W2, b2, W3, b3, LW, Lb):
    raise NotImplementedError("write your pallas kernel here")


# speedup vs baseline: 35.5464x; 35.5464x over previous
import functools

import jax
import jax.numpy as jnp
from jax import lax
from jax.experimental import pallas as pl
from jax.experimental.pallas import tpu as pltpu
from jax.experimental.pallas import tpu_sc as plsc

NC = 2
NS = 16
NW = NC * NS

NP = 102400
DP = 8


def _mesh():
    return plsc.VectorSubcoreMesh(
        core_axis_name="c", subcore_axis_name="s", num_cores=NC, num_subcores=NS
    )


def _make_deg_count(N, E):
    EW = E // NW
    C = 2000
    nch = EW // C
    ZR = NP // NS

    @functools.partial(
        pl.kernel,
        out_type=jax.ShapeDtypeStruct((NC, NP), jnp.float32),
        mesh=_mesh(),
        scratch_types=[
            pltpu.VMEM((C,), jnp.int32),
            pltpu.VMEM((C,), jnp.float32),
            pltpu.VMEM_SHARED((NP,), jnp.float32),
        ],
        compiler_params=pltpu.CompilerParams(use_tc_tiling_on_sc=False),
    )
    def deg_k(dst_hbm, zeros_hbm, out_hbm, dst_v, ones_v, acc_sh):
        cid = lax.axis_index("c")
        sid = lax.axis_index("s")
        wid = cid * NS + sid

        def fill(i, _):
            ones_v[pl.ds(i * 16, 16)] = jnp.full((16,), 1.0, jnp.float32)
            return ()

        lax.fori_loop(0, C // 16, fill, ())
        pltpu.sync_copy(
            zeros_hbm.at[pl.ds(sid * ZR, ZR)], acc_sh.at[pl.ds(sid * ZR, ZR)]
        )
        plsc.subcore_barrier()

        def body(k, _):
            base = wid * EW + k * C
            pltpu.sync_copy(dst_hbm.at[pl.ds(base, C)], dst_v)
            pltpu.sync_copy(ones_v, acc_sh.at[dst_v], add=True)
            return ()

        lax.fori_loop(0, nch, body, ())
        plsc.subcore_barrier()
        pltpu.sync_copy(
            acc_sh.at[pl.ds(sid * ZR, ZR)], out_hbm.at[cid, pl.ds(sid * ZR, ZR)]
        )

    return deg_k


def _make_propagate(N, E):
    EW = E // NW
    C = 2000
    nch = EW // C
    ZR = NP // NS

    @functools.partial(
        pl.kernel,
        out_type=jax.ShapeDtypeStruct((NC, NP, DP), jnp.float32),
        mesh=_mesh(),
        scratch_types=[
            pltpu.VMEM((C,), jnp.int32),
            pltpu.VMEM((C,), jnp.int32),
            pltpu.VMEM((C, DP), jnp.float32),
            pltpu.VMEM_SHARED((NP, DP), jnp.float32),
            pltpu.SemaphoreType.DMA,
        ],
        compiler_params=pltpu.CompilerParams(use_tc_tiling_on_sc=False),
    )
    def prop_k(g_hbm, src_hbm, dst_hbm, zeros_hbm, out_hbm,
               src_v, dst_v, rows_v, acc_sh, sem):
        cid = lax.axis_index("c")
        sid = lax.axis_index("s")
        wid = cid * NS + sid

        pltpu.sync_copy(
            zeros_hbm.at[pl.ds(sid * ZR, ZR)], acc_sh.at[pl.ds(sid * ZR, ZR)]
        )
        plsc.subcore_barrier()

        def body(k, _):
            base = wid * EW + k * C
            pltpu.sync_copy(src_hbm.at[pl.ds(base, C)], src_v)
            pltpu.sync_copy(dst_hbm.at[pl.ds(base, C)], dst_v)
            pltpu.async_copy(g_hbm.at[src_v], rows_v, sem).wait()
            pltpu.sync_copy(rows_v, acc_sh.at[dst_v], add=True)
            return ()

        lax.fori_loop(0, nch, body, ())
        plsc.subcore_barrier()
        pltpu.sync_copy(
            acc_sh.at[pl.ds(sid * ZR, ZR)], out_hbm.at[cid, pl.ds(sid * ZR, ZR)]
        )

    return prop_k


_R = 4000


def _row_spec(d):
    return pl.BlockSpec((_R, d), lambda i: (i, 0))


def _full_spec(shape):
    return pl.BlockSpec(shape, lambda i: (0,) * len(shape))


def _tc_matmul(x, W):
    N, K = x.shape
    Dout = W.shape[1]

    def body(x_ref, w_ref, u_ref):
        u_ref[...] = jnp.dot(x_ref[...], w_ref[...],
                             preferred_element_type=jnp.float32)

    return pl.pallas_call(
        body,
        grid=(N // _R,),
        in_specs=[_row_spec(K), _full_spec((K, Dout))],
        out_specs=_row_spec(Dout),
        out_shape=jax.ShapeDtypeStruct((N, Dout), jnp.float32),
    )(x, W)


def _tc_dis_g(c0, c1, u1):
    N, D = u1.shape

    def body(c0_ref, c1_ref, u_ref, dis_ref, g_ref):
        deg = c0_ref[...] + c1_ref[...] + 1.0
        dis = lax.rsqrt(deg)
        dis_ref[...] = dis
        g_ref[...] = dis * u_ref[...]

    return pl.pallas_call(
        body,
        grid=(N // _R,),
        in_specs=[_row_spec(1), _row_spec(1), _row_spec(D)],
        out_specs=(_row_spec(1), _row_spec(D)),
        out_shape=(
            jax.ShapeDtypeStruct((N, 1), jnp.float32),
            jax.ShapeDtypeStruct((N, D), jnp.float32),
        ),
    )(c0, c1, u1)


def _tc_layer(p0, p1, g, dis, b, W):
    N, D = g.shape
    Dout = W.shape[1]

    def body(p0_ref, p1_ref, g_ref, dis_ref, b_ref, w_ref, out_ref):
        h = jnp.maximum(
            dis_ref[...] * (p0_ref[...] + p1_ref[...] + g_ref[...]) + b_ref[...],
            0.0,
        )
        out_ref[...] = dis_ref[...] * jnp.dot(
            h, w_ref[...], preferred_element_type=jnp.float32
        )

    return pl.pallas_call(
        body,
        grid=(N // _R,),
        in_specs=[_row_spec(D), _row_spec(D), _row_spec(D), _row_spec(1),
                  _full_spec((1, D)), _full_spec((D, Dout))],
        out_specs=_row_spec(Dout),
        out_shape=jax.ShapeDtypeStruct((N, Dout), jnp.float32),
    )(p0, p1, g, dis, b, W)


def _tc_head(p0, p1, g, dis, b, LW, Lb, Dh):
    N, D = g.shape
    Dout = LW.shape[1]

    def body(p0_ref, p1_ref, g_ref, dis_ref, b_ref, lw_ref, lb_ref,
             out_ref, h_ref):
        h = jnp.maximum(
            dis_ref[...] * (p0_ref[...] + p1_ref[...] + g_ref[...]) + b_ref[...],
            0.0,
        )
        h_ref[...] = h[:, :Dh]
        out_ref[...] = (
            jnp.dot(h, lw_ref[...], preferred_element_type=jnp.float32)
            + lb_ref[...]
        )

    return pl.pallas_call(
        body,
        grid=(N // _R,),
        in_specs=[_row_spec(D), _row_spec(D), _row_spec(D), _row_spec(1),
                  _full_spec((1, D)), _full_spec((D, Dout)), _full_spec((1, Dout))],
        out_specs=(_row_spec(Dout), _row_spec(Dh)),
        out_shape=(
            jax.ShapeDtypeStruct((N, Dout), jnp.float32),
            jax.ShapeDtypeStruct((N, Dh), jnp.float32),
        ),
    )(p0, p1, g, dis, b, LW, Lb)


def _pad_w(W):
    return jnp.zeros((W.shape[0] if W.shape[0] > DP else DP, DP),
                     jnp.float32).at[: W.shape[0], : W.shape[1]].set(W)


def _pad_b(b):
    return jnp.zeros((1, DP), jnp.float32).at[0, : b.shape[0]].set(b)


def kernel(x, edge_index, W1, b1, W2, b2, W3, b3, LW, Lb):
    N = x.shape[0]
    E = edge_index.shape[1]

    src = edge_index[0]
    dst = edge_index[1]

    zeros1 = jnp.zeros((NP,), jnp.float32)
    zeros8 = jnp.zeros((NP, DP), jnp.float32)

    W1p = _pad_w(W1)
    W2p = _pad_w(W2)
    W3p = _pad_w(W3)
    LWp = jnp.zeros((DP, LW.shape[1]), jnp.float32).at[: LW.shape[0], :].set(LW)

    cnt = _make_deg_count(N, E)(dst, zeros1)
    c0 = cnt[0, :N, None]
    c1 = cnt[1, :N, None]

    prop = _make_propagate(N, E)

    u1 = _tc_matmul(x, W1p)
    dis, g1 = _tc_dis_g(c0, c1, u1)
    P1 = prop(g1, src, dst, zeros8)
    g2 = _tc_layer(P1[0, :N], P1[1, :N], g1, dis, _pad_b(b1), W2p)

    P2 = prop(g2, src, dst, zeros8)
    g3 = _tc_layer(P2[0, :N], P2[1, :N], g2, dis, _pad_b(b2), W3p)

    P3 = prop(g3, src, dst, zeros8)
    out, h = _tc_head(P3[0, :N], P3[1, :N], g3, dis, _pad_b(b3),
                      LWp, Lb.reshape(1, -1), W3.shape[1])
    return (out, h)
